# baseline (speedup 1.0000x reference)
import functools

import jax
import jax.numpy as jnp
from jax import lax
from jax.experimental import pallas as pl
from jax.experimental.pallas import tpu as pltpu
from jax.experimental.pallas import tpu_sc as plsc

N_NODES = 10000
D = 256
D_EDGE = 16
MEM = 256
B = 1024
NQ = 4096
NB = 20
H = 2
QDIM = 2 * D
DH = QDIM // H

NC = 2
NS = 16
NW = NC * NS
L = 16

EXT_ROWS = N_NODES + B


def _sc_pending_body(mem_hbm, nf_hbm, pend_hbm, h_out, nf_out, idx_v, buf, sem):
    wid = lax.axis_index("s") * NC + lax.axis_index("c")
    per_w = B // NW
    base = pl.multiple_of(wid * per_w, 8)
    pltpu.sync_copy(pend_hbm.at[pl.ds(base, per_w)], idx_v)
    pltpu.async_copy(mem_hbm.at[idx_v], buf, sem).wait()
    pltpu.sync_copy(buf, h_out.at[pl.ds(base, per_w)])
    pltpu.async_copy(nf_hbm.at[idx_v], buf, sem).wait()
    pltpu.sync_copy(buf, nf_out.at[pl.ds(base, per_w)])


def _sc_gather_pending(memory, node_features, pending):
    mesh = plsc.VectorSubcoreMesh(core_axis_name="c", subcore_axis_name="s")
    per_w = B // NW
    f = pl.kernel(
        _sc_pending_body,
        out_type=[
            jax.ShapeDtypeStruct((B, MEM), jnp.float32),
            jax.ShapeDtypeStruct((B, D), jnp.float32),
        ],
        mesh=mesh,
        scratch_types=[
            pltpu.VMEM((per_w,), jnp.int32),
            pltpu.VMEM((per_w, D), jnp.float32),
            pltpu.SemaphoreType.DMA,
        ],
        compiler_params=pltpu.CompilerParams(needs_layout_passes=False, use_tc_tiling_on_sc=False),
    )
    return f(memory, node_features, pending)


def _tc_part1_body(raw_ref, h_ref, nfp_ref, w1_ref, b1_ref, w2_ref, b2_ref,
                   wih_ref, whh_ref, bih_ref, bhh_ref, tb_ref, wqp_ref,
                   out_ref, qbias_ref):
    f32 = jnp.float32
    qbias_ref[...] = jnp.dot(jnp.cos(tb_ref[...]), wqp_ref[...],
                             preferred_element_type=f32)
    raw = raw_ref[...]
    hid = jnp.maximum(
        jnp.dot(raw, w1_ref[...], preferred_element_type=f32) + b1_ref[...], 0.0)
    msg = jnp.dot(hid, w2_ref[...], preferred_element_type=f32) + b2_ref[...]
    h = h_ref[...]
    gi = jnp.dot(msg, wih_ref[...], preferred_element_type=f32) + bih_ref[...]
    gh = jnp.dot(h, whh_ref[...], preferred_element_type=f32) + bhh_ref[...]
    i_r, i_z, i_n = gi[:, :MEM], gi[:, MEM:2 * MEM], gi[:, 2 * MEM:]
    h_r, h_z, h_n = gh[:, :MEM], gh[:, MEM:2 * MEM], gh[:, 2 * MEM:]
    r = jax.nn.sigmoid(i_r + h_r)
    z = jax.nn.sigmoid(i_z + h_z)
    n = jnp.tanh(i_n + r * h_n)
    h_new = (1.0 - z) * n + z * h
    out_ref[...] = h_new + nfp_ref[...]


def _tc_part1(raw, h, nfp, w1, b1, w2, b2, wih_t, whh_t, bih, bhh, tb, wqp):
    return pl.pallas_call(
        _tc_part1_body,
        out_shape=[jax.ShapeDtypeStruct((B, D), jnp.float32),
                   jax.ShapeDtypeStruct((1, QDIM), jnp.float32)],
    )(raw, h, nfp, w1, b1.reshape(1, -1), w2, b2.reshape(1, -1),
      wih_t, whh_t, bih.reshape(1, -1), bhh.reshape(1, -1),
      tb.reshape(1, -1), wqp)


_ADD_BLK = 400
_N_ADD = N_NODES // _ADD_BLK


def _tc_ext_body(a_ref, b_ref, u_ref, o_ref):
    i = pl.program_id(0)

    @pl.when(i < _N_ADD)
    def _():
        o_ref[...] = a_ref[...] + b_ref[...]

    @pl.when(i >= _N_ADD)
    def _():
        o_ref[...] = u_ref[...]


def _tc_ext(memory, node_features, upd):
    nsteps = _N_ADD + (B + _ADD_BLK - 1) // _ADD_BLK
    return pl.pallas_call(
        _tc_ext_body,
        grid=(nsteps,),
        in_specs=[
            pl.BlockSpec((_ADD_BLK, D), lambda i: (jnp.minimum(i, _N_ADD - 1), 0)),
            pl.BlockSpec((_ADD_BLK, D), lambda i: (jnp.minimum(i, _N_ADD - 1), 0)),
            pl.BlockSpec((_ADD_BLK, D), lambda i: (jnp.maximum(i - _N_ADD, 0), 0)),
        ],
        out_specs=pl.BlockSpec((_ADD_BLK, D), lambda i: (i, 0)),
        out_shape=jax.ShapeDtypeStruct((EXT_ROWS, D), jnp.float32),
    )(memory, node_features, upd)


def _sc_main_body(ext_hbm, iota_hbm, pend_hbm, nodes_hbm, nbr_hbm, eidx_hbm,
                  ef_hbm, src_out, nb_out, ef_out,
                  t_v, pend_v, idx_v, tidx_v, rowbuf, efbuf, sem):
    wid = lax.axis_index("s") * NC + lax.axis_index("c")
    lanes = lax.iota(jnp.int32, L)

    pltpu.sync_copy(iota_hbm, t_v)
    pltpu.sync_copy(pend_hbm, pend_v.at[pl.ds(0, B)])
    pend_v[pl.ds(B, L)] = jnp.full((L,), -1, jnp.int32)
    for i in range(B // L):
        idx = pend_v[pl.ds(i * L, L)]
        nxt = plsc.load_gather(pend_v, [lanes + (i * L + 1)])
        keep = idx != nxt
        vals = lanes + (N_NODES + i * L)
        plsc.store_scatter(t_v, [idx], vals, mask=keep)

    sbase = pl.multiple_of(wid * (NQ // NW), 8)
    pltpu.sync_copy(nodes_hbm.at[pl.ds(sbase, NQ // NW)], idx_v)
    for j in range(128 // L):
        v = idx_v[pl.ds(j * L, L)]
        tidx_v[pl.ds(j * L, L)] = plsc.load_gather(t_v, [v])
    pltpu.async_copy(ext_hbm.at[tidx_v], rowbuf, sem).wait()
    pltpu.sync_copy(rowbuf, src_out.at[pl.ds(sbase, NQ // NW)])

    per_w = (NQ * NB) // NW
    nchunks = per_w // 128

    def nb_chunk(c, _):
        base = pl.multiple_of(wid * per_w + c * 128, 8)
        pltpu.sync_copy(nbr_hbm.at[pl.ds(base, 128)], idx_v)
        for j in range(128 // L):
            v = idx_v[pl.ds(j * L, L)]
            tidx_v[pl.ds(j * L, L)] = plsc.load_gather(t_v, [v])
        pltpu.async_copy(ext_hbm.at[tidx_v], rowbuf, sem).wait()
        pltpu.sync_copy(rowbuf, nb_out.at[pl.ds(base, 128)])
        return 0

    lax.fori_loop(0, nchunks, nb_chunk, 0)

    def ef_chunk(c, _):
        base = pl.multiple_of(wid * per_w + c * 128, 8)
        pltpu.sync_copy(eidx_hbm.at[pl.ds(base, 128)], idx_v)
        pltpu.async_copy(ef_hbm.at[idx_v], efbuf, sem).wait()
        pltpu.sync_copy(efbuf, ef_out.at[pl.ds(base, 128)])
        return 0

    lax.fori_loop(0, nchunks, ef_chunk, 0)


def _sc_gather_main(ext, iota, pending, nodes, nbr_flat, eidx_flat, edge_features):
    mesh = plsc.VectorSubcoreMesh(core_axis_name="c", subcore_axis_name="s")
    f = pl.kernel(
        _sc_main_body,
        out_type=[
            jax.ShapeDtypeStruct((NQ, D), jnp.float32),
            jax.ShapeDtypeStruct((NQ * NB, D), jnp.float32),
            jax.ShapeDtypeStruct((NQ * NB, D_EDGE), jnp.float32),
        ],
        mesh=mesh,
        scratch_types=[
            pltpu.VMEM((N_NODES,), jnp.int32),
            pltpu.VMEM((B + L,), jnp.int32),
            pltpu.VMEM((128,), jnp.int32),
            pltpu.VMEM((128,), jnp.int32),
            pltpu.VMEM((128, D), jnp.float32),
            pltpu.VMEM((128, D_EDGE), jnp.float32),
            pltpu.SemaphoreType.DMA,
        ],
        compiler_params=pltpu.CompilerParams(needs_layout_passes=False, use_tc_tiling_on_sc=False),
    )
    return f(ext, iota, pending, nodes, nbr_flat, eidx_flat, edge_features)


_COS_COEFFS = (1.0, -0.5, 0.0416666641831398, -0.0013888858957216144,
               2.4800418032100424e-05, -2.753243677489081e-07,
               2.058421877393357e-09, -9.662048938707812e-12)
_INV2PI = 0.15915494309189535
_TWOPI_HI = 6.28125
_TWOPI_LO = 0.0019353071795864769


def _fast_cos(x):
    n = jnp.floor(x * _INV2PI + 0.5)
    r = x - n * _TWOPI_HI
    r = r - n * _TWOPI_LO
    r2 = r * r
    acc = jnp.full_like(x, _COS_COEFFS[-1])
    for c in _COS_COEFFS[-2::-1]:
        acc = acc * r2 + c
    return acc


def _tc_attn_body(src_ref, nb_ref, ef_ref, dt_ref, nbr_ref, tw_ref, tb_ref,
                  wqm_ref, qb_ref, wkm_ref, wkp_ref, wke_ref,
                  wvm_ref, wvp_ref, wve_ref, wo_ref,
                  f1a_ref, f1c_ref, f1b_ref, f2w_ref, f2b_ref, out_ref):
    f32 = jnp.float32
    dot = functools.partial(jnp.dot, preferred_element_type=f32)
    tw = tw_ref[...]
    tb = tb_ref[...]
    src = src_ref[...]

    q = dot(src, wqm_ref[...]) + qb_ref[...]

    ks = []
    vs = []
    for n in range(NB):
        nb_n = nb_ref[n]
        ef_n = ef_ref[n]
        phi_n = _fast_cos(dt_ref[n] * tw + tb)
        k_n = dot(nb_n, wkm_ref[...]) + dot(phi_n, wkp_ref[...]) + dot(ef_n, wke_ref[...])
        v_n = dot(nb_n, wvm_ref[...]) + dot(phi_n, wvp_ref[...]) + dot(ef_n, wve_ref[...])
        ks.append(k_n)
        vs.append(v_n)

    scale = 1.0 / (DH ** 0.5)
    outs = []
    for h in range(H):
        qh = q[:, h * DH:(h + 1) * DH]
        cols = []
        for n in range(NB):
            s_n = jnp.sum(qh * ks[n][:, h * DH:(h + 1) * DH], axis=1,
                          keepdims=True) * scale
            s_n = jnp.where(nbr_ref[n] == 0, -1e9, s_n)
            cols.append(s_n)
        s = jnp.concatenate(cols, axis=1)
        s = s - jnp.max(s, axis=1, keepdims=True)
        e = jnp.exp(s)
        a = e / jnp.sum(e, axis=1, keepdims=True)
        o_h = jnp.zeros_like(qh)
        for n in range(NB):
            o_h = o_h + a[:, n:n + 1] * vs[n][:, h * DH:(h + 1) * DH]
        outs.append(o_h)
    att = jnp.concatenate(outs, axis=1)
    out = dot(att, wo_ref[...])
    merged = jnp.maximum(
        dot(out, f1a_ref[...]) + dot(src, f1c_ref[...]) + f1b_ref[...], 0.0)
    out_ref[...] = dot(merged, f2w_ref[...]) + f2b_ref[...]


def _tc_attn(src_feat, nb3, ef3, dt3, nbr3, time_w, time_b, Wq, qbias, Wk, Wv,
             Wo, fc1_w, fc1_b, fc2_w, fc2_b):
    BQ = 128
    grid = (NQ // BQ,)
    const = lambda shape: pl.BlockSpec(shape, lambda i: tuple(0 for _ in shape))
    in_specs = [
        pl.BlockSpec((BQ, D), lambda i: (i, 0)),
        pl.BlockSpec((NB, BQ, D), lambda i: (0, i, 0)),
        pl.BlockSpec((NB, BQ, D_EDGE), lambda i: (0, i, 0)),
        pl.BlockSpec((NB, BQ, 1), lambda i: (0, i, 0)),
        pl.BlockSpec((NB, BQ, 1), lambda i: (0, i, 0)),
        const((1, D)),
        const((1, D)),
        const((D, QDIM)), const((1, QDIM)),
        const((D, QDIM)), const((D, QDIM)), const((D_EDGE, QDIM)),
        const((D, QDIM)), const((D, QDIM)), const((D_EDGE, QDIM)),
        const((QDIM, QDIM)),
        const((QDIM, D)), const((D, D)), const((1, D)),
        const((D, D)), const((1, D)),
    ]
    return pl.pallas_call(
        _tc_attn_body,
        grid=grid,
        in_specs=in_specs,
        out_specs=pl.BlockSpec((BQ, D), lambda i: (i, 0)),
        out_shape=jax.ShapeDtypeStruct((NQ, D), jnp.float32),
    )(src_feat, nb3, ef3, dt3, nbr3,
      time_w.reshape(1, D), time_b.reshape(1, D),
      Wq[:D], qbias, Wk[:D], Wk[D:2 * D], Wk[2 * D:],
      Wv[:D], Wv[D:2 * D], Wv[2 * D:], Wo,
      fc1_w[:QDIM], fc1_w[QDIM:], fc1_b.reshape(1, D),
      fc2_w, fc2_b.reshape(1, D))


def kernel(node_features, edge_features, memory, time_w, time_b, msg_w1,
           msg_b1, msg_w2, msg_b2, gru_wih, gru_whh, gru_bih, gru_bhh, Wq, Wk,
           Wv, Wo, fc1_w, fc1_b, fc2_w, fc2_b, edge_times, neighbor_times,
           pending_msg_raw, source_nodes, destination_nodes, p_pos_nodes,
           p_neg_nodes, edge_idxs, neighbors, neighbor_edge_idxs,
           pending_msg_nodes):
    pending = pending_msg_nodes.astype(jnp.int32)

    h, nfp = _sc_gather_pending(memory, node_features, pending)
    upd, qbias = _tc_part1(pending_msg_raw, h, nfp, msg_w1, msg_b1, msg_w2,
                           msg_b2, gru_wih.T, gru_whh.T, gru_bih, gru_bhh,
                           time_b, Wq[D:])
    ext = _tc_ext(memory, node_features, upd)

    nodes = jnp.concatenate(
        [source_nodes, destination_nodes, p_pos_nodes, p_neg_nodes]
    ).astype(jnp.int32)
    nbr_flat = neighbors.T.reshape(-1).astype(jnp.int32)
    eidx_flat = neighbor_edge_idxs.T.reshape(-1).astype(jnp.int32)
    iota = jnp.arange(N_NODES, dtype=jnp.int32)

    src_feat, nb_flat, ef_flat = _sc_gather_main(
        ext, iota, pending, nodes, nbr_flat, eidx_flat, edge_features)

    ts = jnp.tile(edge_times, 4)
    dt3 = (ts[None, :] - neighbor_times.T)[..., None]
    nb3 = nb_flat.reshape(NB, NQ, D)
    ef3 = ef_flat.reshape(NB, NQ, D_EDGE)
    nbr3 = neighbors.T[..., None].astype(jnp.int32)

    return _tc_attn(src_feat, nb3, ef3, dt3, nbr3, time_w, time_b,
                    Wq, qbias, Wk, Wv, Wo, fc1_w, fc1_b, fc2_w, fc2_b)


# speedup vs baseline: 1.3844x; 1.3844x over previous
import functools

import jax
import jax.numpy as jnp
from jax import lax
from jax.experimental import pallas as pl
from jax.experimental.pallas import tpu as pltpu
from jax.experimental.pallas import tpu_sc as plsc

N_NODES = 10000
D = 256
D_EDGE = 16
MEM = 256
B = 1024
NQ = 4096
NB = 20
H = 2
QDIM = 2 * D
DH = QDIM // H

NC = 2
NS = 16
NW = NC * NS
L = 16

EXT_ROWS = N_NODES + B


def _sc_pending_body(mem_hbm, nf_hbm, pend_hbm, h_out, nf_out, pend_v, buf, sem):
    wid = lax.axis_index("s") * NC + lax.axis_index("c")
    per_w = B // NW
    base = pl.multiple_of(wid * per_w, 8)
    pltpu.sync_copy(pend_hbm, pend_v)
    idx = pend_v.at[pl.ds(base, per_w)]
    pltpu.async_copy(mem_hbm.at[idx], buf, sem).wait()
    pltpu.sync_copy(buf, h_out.at[pl.ds(base, per_w)])
    pltpu.async_copy(nf_hbm.at[idx], buf, sem).wait()
    pltpu.sync_copy(buf, nf_out.at[pl.ds(base, per_w)])


def _sc_gather_pending(memory, node_features, pending):
    mesh = plsc.VectorSubcoreMesh(core_axis_name="c", subcore_axis_name="s")
    per_w = B // NW
    f = pl.kernel(
        _sc_pending_body,
        out_type=[
            jax.ShapeDtypeStruct((B, MEM), jnp.float32),
            jax.ShapeDtypeStruct((B, D), jnp.float32),
        ],
        mesh=mesh,
        scratch_types=[
            pltpu.VMEM((B,), jnp.int32),
            pltpu.VMEM((per_w, D), jnp.float32),
            pltpu.SemaphoreType.DMA,
        ],
        compiler_params=pltpu.CompilerParams(needs_layout_passes=False,
                                             use_tc_tiling_on_sc=True),
    )
    return f(memory, node_features, pending)


def _tc_part1_body(raw_ref, h_ref, nfp_ref, w1_ref, b1_ref, w2_ref, b2_ref,
                   wih_ref, whh_ref, bih_ref, bhh_ref, tb_ref, wqp_ref,
                   out_ref, qbias_ref):
    f32 = jnp.float32
    qbias_ref[...] = jnp.dot(jnp.cos(tb_ref[...]), wqp_ref[...],
                             preferred_element_type=f32)
    raw = raw_ref[...]
    hid = jnp.maximum(
        jnp.dot(raw, w1_ref[...], preferred_element_type=f32) + b1_ref[...], 0.0)
    msg = jnp.dot(hid, w2_ref[...], preferred_element_type=f32) + b2_ref[...]
    h = h_ref[...]
    gi = jnp.dot(msg, wih_ref[...], preferred_element_type=f32) + bih_ref[...]
    gh = jnp.dot(h, whh_ref[...], preferred_element_type=f32) + bhh_ref[...]
    i_r, i_z, i_n = gi[:, :MEM], gi[:, MEM:2 * MEM], gi[:, 2 * MEM:]
    h_r, h_z, h_n = gh[:, :MEM], gh[:, MEM:2 * MEM], gh[:, 2 * MEM:]
    r = jax.nn.sigmoid(i_r + h_r)
    z = jax.nn.sigmoid(i_z + h_z)
    n = jnp.tanh(i_n + r * h_n)
    h_new = (1.0 - z) * n + z * h
    out_ref[...] = h_new + nfp_ref[...]


def _tc_part1(raw, h, nfp, w1, b1, w2, b2, wih_t, whh_t, bih, bhh, tb, wqp):
    return pl.pallas_call(
        _tc_part1_body,
        out_shape=[jax.ShapeDtypeStruct((B, D), jnp.float32),
                   jax.ShapeDtypeStruct((1, QDIM), jnp.float32)],
    )(raw, h, nfp, w1, b1.reshape(1, -1), w2, b2.reshape(1, -1),
      wih_t, whh_t, bih.reshape(1, -1), bhh.reshape(1, -1),
      tb.reshape(1, -1), wqp)


_ADD_BLK = 1000
_N_ADD = N_NODES // _ADD_BLK


def _tc_ext_body(a_ref, b_ref, u_ref, o_ref):
    i = pl.program_id(0)

    @pl.when(i < _N_ADD)
    def _():
        o_ref[...] = a_ref[...] + b_ref[...]

    @pl.when(i >= _N_ADD)
    def _():
        o_ref[...] = u_ref[...]


def _tc_ext(memory, node_features, upd):
    nsteps = _N_ADD + (B + _ADD_BLK - 1) // _ADD_BLK
    return pl.pallas_call(
        _tc_ext_body,
        grid=(nsteps,),
        in_specs=[
            pl.BlockSpec((_ADD_BLK, D), lambda i: (jnp.minimum(i, _N_ADD - 1), 0)),
            pl.BlockSpec((_ADD_BLK, D), lambda i: (jnp.minimum(i, _N_ADD - 1), 0)),
            pl.BlockSpec((_ADD_BLK, D), lambda i: (jnp.maximum(i - _N_ADD, 0), 0)),
        ],
        out_specs=pl.BlockSpec((_ADD_BLK, D), lambda i: (i, 0)),
        out_shape=jax.ShapeDtypeStruct((EXT_ROWS, D), jnp.float32),
    )(memory, node_features, upd)


def _sc_main_body(ext_hbm, iota_hbm, pend_hbm, nodes_hbm, nbr_hbm,
                  src_out, nb_out,
                  t_v, pend_v, idx_v, tidx_v, rowbuf, sem):
    wid = lax.axis_index("s") * NC + lax.axis_index("c")
    lanes = lax.iota(jnp.int32, L)

    pltpu.sync_copy(iota_hbm, t_v)
    pltpu.sync_copy(pend_hbm, pend_v.at[pl.ds(0, B)])
    pend_v[pl.ds(B, L)] = jnp.full((L,), -1, jnp.int32)
    for i in range(B // L):
        idx = pend_v[pl.ds(i * L, L)]
        nxt = plsc.load_gather(pend_v, [lanes + (i * L + 1)])
        keep = idx != nxt
        vals = lanes + (N_NODES + i * L)
        plsc.store_scatter(t_v, [idx], vals, mask=keep)

    sbase = pl.multiple_of(wid * (NQ // NW), 8)
    pltpu.sync_copy(nodes_hbm.at[pl.ds(sbase, NQ // NW)], idx_v)
    for j in range(128 // L):
        v = idx_v[pl.ds(j * L, L)]
        tidx_v[pl.ds(j * L, L)] = plsc.load_gather(t_v, [v])
    pltpu.async_copy(ext_hbm.at[tidx_v], rowbuf, sem).wait()
    pltpu.sync_copy(rowbuf, src_out.at[pl.ds(sbase, NQ // NW)])

    per_w = (NQ * NB) // NW
    nchunks = per_w // 128

    def nb_chunk(c, _):
        base = pl.multiple_of(wid * per_w + c * 128, 8)
        pltpu.sync_copy(nbr_hbm.at[pl.ds(base, 128)], idx_v)
        for j in range(128 // L):
            v = idx_v[pl.ds(j * L, L)]
            tidx_v[pl.ds(j * L, L)] = plsc.load_gather(t_v, [v])
        pltpu.async_copy(ext_hbm.at[tidx_v], rowbuf, sem).wait()
        pltpu.sync_copy(rowbuf, nb_out.at[pl.ds(base, 128)])
        return 0

    lax.fori_loop(0, nchunks, nb_chunk, 0)


def _sc_gather_main(ext, iota, pending, nodes, nbr_flat):
    mesh = plsc.VectorSubcoreMesh(core_axis_name="c", subcore_axis_name="s")
    f = pl.kernel(
        _sc_main_body,
        out_type=[
            jax.ShapeDtypeStruct((NQ, D), jnp.float32),
            jax.ShapeDtypeStruct((NQ * NB, D), jnp.float32),
        ],
        mesh=mesh,
        scratch_types=[
            pltpu.VMEM((N_NODES,), jnp.int32),
            pltpu.VMEM((B + L,), jnp.int32),
            pltpu.VMEM((128,), jnp.int32),
            pltpu.VMEM((128,), jnp.int32),
            pltpu.VMEM((128, D), jnp.float32),
            pltpu.SemaphoreType.DMA,
        ],
        compiler_params=pltpu.CompilerParams(needs_layout_passes=False,
                                             use_tc_tiling_on_sc=True),
    )
    return f(ext, iota, pending, nodes, nbr_flat)


def _sc_ef_body(eidx_hbm, ef_hbm, ef_out, idx_v, efbuf, sem):
    wid = lax.axis_index("s") * NC + lax.axis_index("c")
    per_w = (NQ * NB) // NW

    def ef_chunk(c, _):
        base = pl.multiple_of(wid * per_w + c * 128, 8)
        pltpu.sync_copy(eidx_hbm.at[pl.ds(base, 128)], idx_v)
        pltpu.async_copy(ef_hbm.at[idx_v], efbuf, sem).wait()
        pltpu.sync_copy(efbuf, ef_out.at[pl.ds(base, 128)])
        return 0

    lax.fori_loop(0, per_w // 128, ef_chunk, 0)


def _sc_gather_ef(eidx_flat, edge_features):
    mesh = plsc.VectorSubcoreMesh(core_axis_name="c", subcore_axis_name="s")
    f = pl.kernel(
        _sc_ef_body,
        out_type=jax.ShapeDtypeStruct((NQ * NB, D_EDGE), jnp.float32),
        mesh=mesh,
        scratch_types=[
            pltpu.VMEM((128,), jnp.int32),
            pltpu.VMEM((128, D_EDGE), jnp.float32),
            pltpu.SemaphoreType.DMA,
        ],
        compiler_params=pltpu.CompilerParams(needs_layout_passes=False,
                                             use_tc_tiling_on_sc=False),
    )
    return f(eidx_flat, edge_features)


_COS_COEFFS = (1.0, -0.5, 0.0416666641831398, -0.0013888858957216144,
               2.4800418032100424e-05, -2.753243677489081e-07,
               2.058421877393357e-09, -9.662048938707812e-12)
_INV2PI = 0.15915494309189535
_TWOPI_HI = 6.28125
_TWOPI_LO = 0.0019353071795864769


def _fast_cos(x):
    n = jnp.floor(x * _INV2PI + 0.5)
    r = x - n * _TWOPI_HI
    r = r - n * _TWOPI_LO
    r2 = r * r
    acc = jnp.full_like(x, _COS_COEFFS[-1])
    for c in _COS_COEFFS[-2::-1]:
        acc = acc * r2 + c
    return acc


def _tc_attn_body(src_ref, nb_ref, ef_ref, dt_ref, nbr_ref, tw_ref, tb_ref,
                  wqm_ref, qb_ref, wkm_ref, wkp_ref, wke_ref,
                  wvm_ref, wvp_ref, wve_ref, wo_ref,
                  f1a_ref, f1c_ref, f1b_ref, f2w_ref, f2b_ref, out_ref):
    f32 = jnp.float32
    dot = functools.partial(jnp.dot, preferred_element_type=f32)
    tw = tw_ref[...]
    tb = tb_ref[...]
    src = src_ref[...]

    q = dot(src, wqm_ref[...]) + qb_ref[...]

    ks = []
    vs = []
    for n in range(NB):
        nb_n = nb_ref[n]
        ef_n = ef_ref[n]
        phi_n = _fast_cos(dt_ref[:, n:n + 1] * tw + tb)
        k_n = dot(nb_n, wkm_ref[...]) + dot(phi_n, wkp_ref[...]) + dot(ef_n, wke_ref[...])
        v_n = dot(nb_n, wvm_ref[...]) + dot(phi_n, wvp_ref[...]) + dot(ef_n, wve_ref[...])
        ks.append(k_n)
        vs.append(v_n)

    scale = 1.0 / (DH ** 0.5)
    outs = []
    for h in range(H):
        qh = q[:, h * DH:(h + 1) * DH]
        cols = []
        for n in range(NB):
            s_n = jnp.sum(qh * ks[n][:, h * DH:(h + 1) * DH], axis=1,
                          keepdims=True) * scale
            s_n = jnp.where(nbr_ref[:, n:n + 1] == 0, -1e9, s_n)
            cols.append(s_n)
        s = jnp.concatenate(cols, axis=1)
        s = s - jnp.max(s, axis=1, keepdims=True)
        e = jnp.exp(s)
        a = e / jnp.sum(e, axis=1, keepdims=True)
        o_h = jnp.zeros_like(qh)
        for n in range(NB):
            o_h = o_h + a[:, n:n + 1] * vs[n][:, h * DH:(h + 1) * DH]
        outs.append(o_h)
    att = jnp.concatenate(outs, axis=1)
    out = dot(att, wo_ref[...])
    merged = jnp.maximum(
        dot(out, f1a_ref[...]) + dot(src, f1c_ref[...]) + f1b_ref[...], 0.0)
    out_ref[...] = dot(merged, f2w_ref[...]) + f2b_ref[...]


def _tc_attn(src_feat, nb3, ef3, dt3, nbr3, time_w, time_b, Wq, qbias, Wk, Wv,
             Wo, fc1_w, fc1_b, fc2_w, fc2_b):
    BQ = 128
    grid = (NQ // BQ,)
    const = lambda shape: pl.BlockSpec(shape, lambda i: tuple(0 for _ in shape))
    in_specs = [
        pl.BlockSpec((BQ, D), lambda i: (i, 0)),
        pl.BlockSpec((NB, BQ, D), lambda i: (0, i, 0)),
        pl.BlockSpec((NB, BQ, D_EDGE), lambda i: (0, i, 0)),
        pl.BlockSpec((BQ, NB), lambda i: (i, 0)),
        pl.BlockSpec((BQ, NB), lambda i: (i, 0)),
        const((1, D)),
        const((1, D)),
        const((D, QDIM)), const((1, QDIM)),
        const((D, QDIM)), const((D, QDIM)), const((D_EDGE, QDIM)),
        const((D, QDIM)), const((D, QDIM)), const((D_EDGE, QDIM)),
        const((QDIM, QDIM)),
        const((QDIM, D)), const((D, D)), const((1, D)),
        const((D, D)), const((1, D)),
    ]
    return pl.pallas_call(
        _tc_attn_body,
        grid=grid,
        in_specs=in_specs,
        out_specs=pl.BlockSpec((BQ, D), lambda i: (i, 0)),
        out_shape=jax.ShapeDtypeStruct((NQ, D), jnp.float32),
    )(src_feat, nb3, ef3, dt3, nbr3,
      time_w.reshape(1, D), time_b.reshape(1, D),
      Wq[:D], qbias, Wk[:D], Wk[D:2 * D], Wk[2 * D:],
      Wv[:D], Wv[D:2 * D], Wv[2 * D:], Wo,
      fc1_w[:QDIM], fc1_w[QDIM:], fc1_b.reshape(1, D),
      fc2_w, fc2_b.reshape(1, D))


def kernel(node_features, edge_features, memory, time_w, time_b, msg_w1,
           msg_b1, msg_w2, msg_b2, gru_wih, gru_whh, gru_bih, gru_bhh, Wq, Wk,
           Wv, Wo, fc1_w, fc1_b, fc2_w, fc2_b, edge_times, neighbor_times,
           pending_msg_raw, source_nodes, destination_nodes, p_pos_nodes,
           p_neg_nodes, edge_idxs, neighbors, neighbor_edge_idxs,
           pending_msg_nodes):
    pending = pending_msg_nodes.astype(jnp.int32)

    h, nfp = _sc_gather_pending(memory, node_features, pending)
    upd, qbias = _tc_part1(pending_msg_raw, h, nfp, msg_w1, msg_b1, msg_w2,
                           msg_b2, gru_wih.T, gru_whh.T, gru_bih, gru_bhh,
                           time_b, Wq[D:])
    ext = _tc_ext(memory, node_features, upd)

    nodes = jnp.concatenate(
        [source_nodes, destination_nodes, p_pos_nodes, p_neg_nodes]
    ).astype(jnp.int32)
    nbr_flat = neighbors.T.reshape(-1).astype(jnp.int32)
    eidx_flat = neighbor_edge_idxs.T.reshape(-1).astype(jnp.int32)
    iota = jnp.arange(N_NODES, dtype=jnp.int32)

    src_feat, nb_flat = _sc_gather_main(ext, iota, pending, nodes, nbr_flat)
    ef_flat = _sc_gather_ef(eidx_flat, edge_features)

    ts = jnp.tile(edge_times, 4)
    dt2 = ts[:, None] - neighbor_times
    nb3 = nb_flat.reshape(NB, NQ, D)
    ef3 = ef_flat.reshape(NB, NQ, D_EDGE)
    nbr2 = neighbors.astype(jnp.int32)

    return _tc_attn(src_feat, nb3, ef3, dt2, nbr2, time_w, time_b,
                    Wq, qbias, Wk, Wv, Wo, fc1_w, fc1_b, fc2_w, fc2_b)


# speedup vs baseline: 1.3946x; 1.0074x over previous
import functools

import jax
import jax.numpy as jnp
from jax import lax
from jax.experimental import pallas as pl
from jax.experimental.pallas import tpu as pltpu
from jax.experimental.pallas import tpu_sc as plsc

N_NODES = 10000
D = 256
D_EDGE = 16
MEM = 256
B = 1024
NQ = 4096
NB = 20
H = 2
QDIM = 2 * D
DH = QDIM // H

NC = 2
NS = 16
NW = NC * NS
L = 16

EXT_ROWS = N_NODES + B


def _sc_pending_body(mem_hbm, nf_hbm, pend_hbm, h_out, nf_out, pend_v, buf, sem):
    wid = lax.axis_index("s") * NC + lax.axis_index("c")
    per_w = B // NW
    base = pl.multiple_of(wid * per_w, 8)
    pltpu.sync_copy(pend_hbm, pend_v)
    idx = pend_v.at[pl.ds(base, per_w)]
    pltpu.async_copy(mem_hbm.at[idx], buf, sem).wait()
    pltpu.sync_copy(buf, h_out.at[pl.ds(base, per_w)])
    pltpu.async_copy(nf_hbm.at[idx], buf, sem).wait()
    pltpu.sync_copy(buf, nf_out.at[pl.ds(base, per_w)])


def _sc_gather_pending(memory, node_features, pending):
    mesh = plsc.VectorSubcoreMesh(core_axis_name="c", subcore_axis_name="s")
    per_w = B // NW
    f = pl.kernel(
        _sc_pending_body,
        out_type=[
            jax.ShapeDtypeStruct((B, MEM), jnp.float32),
            jax.ShapeDtypeStruct((B, D), jnp.float32),
        ],
        mesh=mesh,
        scratch_types=[
            pltpu.VMEM((B,), jnp.int32),
            pltpu.VMEM((per_w, D), jnp.float32),
            pltpu.SemaphoreType.DMA,
        ],
        compiler_params=pltpu.CompilerParams(needs_layout_passes=False,
                                             use_tc_tiling_on_sc=True),
    )
    return f(memory, node_features, pending)


def _tc_part1_body(raw_ref, h_ref, nfp_ref, w1_ref, b1_ref, w2_ref, b2_ref,
                   wih_ref, whh_ref, bih_ref, bhh_ref, tb_ref, wqp_ref,
                   out_ref, qbias_ref):
    f32 = jnp.float32
    qbias_ref[...] = jnp.dot(jnp.cos(tb_ref[...]), wqp_ref[...],
                             preferred_element_type=f32)
    raw = raw_ref[...]
    hid = jnp.maximum(
        jnp.dot(raw, w1_ref[...], preferred_element_type=f32) + b1_ref[...], 0.0)
    msg = jnp.dot(hid, w2_ref[...], preferred_element_type=f32) + b2_ref[...]
    h = h_ref[...]
    gi = jnp.dot(msg, wih_ref[...], preferred_element_type=f32) + bih_ref[...]
    gh = jnp.dot(h, whh_ref[...], preferred_element_type=f32) + bhh_ref[...]
    i_r, i_z, i_n = gi[:, :MEM], gi[:, MEM:2 * MEM], gi[:, 2 * MEM:]
    h_r, h_z, h_n = gh[:, :MEM], gh[:, MEM:2 * MEM], gh[:, 2 * MEM:]
    r = jax.nn.sigmoid(i_r + h_r)
    z = jax.nn.sigmoid(i_z + h_z)
    n = jnp.tanh(i_n + r * h_n)
    h_new = (1.0 - z) * n + z * h
    out_ref[...] = h_new + nfp_ref[...]


def _tc_part1(raw, h, nfp, w1, b1, w2, b2, wih_t, whh_t, bih, bhh, tb, wqp):
    return pl.pallas_call(
        _tc_part1_body,
        out_shape=[jax.ShapeDtypeStruct((B, D), jnp.float32),
                   jax.ShapeDtypeStruct((1, QDIM), jnp.float32)],
    )(raw, h, nfp, w1, b1.reshape(1, -1), w2, b2.reshape(1, -1),
      wih_t, whh_t, bih.reshape(1, -1), bhh.reshape(1, -1),
      tb.reshape(1, -1), wqp)


_ADD_BLK = 1000
_N_ADD = N_NODES // _ADD_BLK


def _tc_ext_body(a_ref, b_ref, u_ref, o_ref):
    i = pl.program_id(0)

    @pl.when(i < _N_ADD)
    def _():
        o_ref[...] = a_ref[...] + b_ref[...]

    @pl.when(i >= _N_ADD)
    def _():
        o_ref[...] = u_ref[...]


def _tc_ext(memory, node_features, upd):
    nsteps = _N_ADD + (B + _ADD_BLK - 1) // _ADD_BLK
    return pl.pallas_call(
        _tc_ext_body,
        grid=(nsteps,),
        in_specs=[
            pl.BlockSpec((_ADD_BLK, D), lambda i: (jnp.minimum(i, _N_ADD - 1), 0)),
            pl.BlockSpec((_ADD_BLK, D), lambda i: (jnp.minimum(i, _N_ADD - 1), 0)),
            pl.BlockSpec((_ADD_BLK, D), lambda i: (jnp.maximum(i - _N_ADD, 0), 0)),
        ],
        out_specs=pl.BlockSpec((_ADD_BLK, D), lambda i: (i, 0)),
        out_shape=jax.ShapeDtypeStruct((EXT_ROWS, D), jnp.float32),
    )(memory, node_features, upd)


def _sc_main_body(ext_hbm, iota_hbm, pend_hbm, nodes_hbm, nbr_hbm,
                  src_out, nb_out,
                  t_v, pend_v, idx_v, tidx_v, rowbuf, sem):
    wid = lax.axis_index("s") * NC + lax.axis_index("c")
    lanes = lax.iota(jnp.int32, L)

    pltpu.sync_copy(iota_hbm, t_v)
    pltpu.sync_copy(pend_hbm, pend_v.at[pl.ds(0, B)])
    pend_v[pl.ds(B, L)] = jnp.full((L,), -1, jnp.int32)
    for i in range(B // L):
        idx = pend_v[pl.ds(i * L, L)]
        nxt = plsc.load_gather(pend_v, [lanes + (i * L + 1)])
        keep = idx != nxt
        vals = lanes + (N_NODES + i * L)
        plsc.store_scatter(t_v, [idx], vals, mask=keep)

    sbase = pl.multiple_of(wid * (NQ // NW), 8)
    pltpu.sync_copy(nodes_hbm.at[pl.ds(sbase, NQ // NW)], idx_v)
    for j in range(128 // L):
        v = idx_v[pl.ds(j * L, L)]
        tidx_v[pl.ds(j * L, L)] = plsc.load_gather(t_v, [v])
    pltpu.async_copy(ext_hbm.at[tidx_v], rowbuf, sem).wait()
    pltpu.sync_copy(rowbuf, src_out.at[pl.ds(sbase, NQ // NW)])

    per_w = (NQ * NB) // NW
    nchunks = per_w // 128

    def nb_chunk(c, _):
        base = pl.multiple_of(wid * per_w + c * 128, 8)
        pltpu.sync_copy(nbr_hbm.at[pl.ds(base, 128)], idx_v)
        for j in range(128 // L):
            v = idx_v[pl.ds(j * L, L)]
            tidx_v[pl.ds(j * L, L)] = plsc.load_gather(t_v, [v])
        pltpu.async_copy(ext_hbm.at[tidx_v], rowbuf, sem).wait()
        pltpu.sync_copy(rowbuf, nb_out.at[pl.ds(base, 128)])
        return 0

    lax.fori_loop(0, nchunks, nb_chunk, 0)


def _sc_gather_main(ext, iota, pending, nodes, nbr_flat):
    mesh = plsc.VectorSubcoreMesh(core_axis_name="c", subcore_axis_name="s")
    f = pl.kernel(
        _sc_main_body,
        out_type=[
            jax.ShapeDtypeStruct((NQ, D), jnp.float32),
            jax.ShapeDtypeStruct((NQ * NB, D), jnp.float32),
        ],
        mesh=mesh,
        scratch_types=[
            pltpu.VMEM((N_NODES,), jnp.int32),
            pltpu.VMEM((B + L,), jnp.int32),
            pltpu.VMEM((128,), jnp.int32),
            pltpu.VMEM((128,), jnp.int32),
            pltpu.VMEM((128, D), jnp.float32),
            pltpu.SemaphoreType.DMA,
        ],
        compiler_params=pltpu.CompilerParams(needs_layout_passes=False,
                                             use_tc_tiling_on_sc=True),
    )
    return f(ext, iota, pending, nodes, nbr_flat)


def _sc_ef_body(eidx_hbm, ef_hbm, ef_out, idx_v, efbuf, sem):
    wid = lax.axis_index("s") * NC + lax.axis_index("c")
    per_w = (NQ * NB) // NW

    def ef_chunk(c, _):
        base = pl.multiple_of(wid * per_w + c * 128, 8)
        pltpu.sync_copy(eidx_hbm.at[pl.ds(base, 128)], idx_v)
        pltpu.async_copy(ef_hbm.at[idx_v], efbuf, sem).wait()
        pltpu.sync_copy(efbuf, ef_out.at[pl.ds(base, 128)])
        return 0

    lax.fori_loop(0, per_w // 128, ef_chunk, 0)


def _sc_gather_ef(eidx_flat, edge_features):
    mesh = plsc.VectorSubcoreMesh(core_axis_name="c", subcore_axis_name="s")
    f = pl.kernel(
        _sc_ef_body,
        out_type=jax.ShapeDtypeStruct((NQ * NB, D_EDGE), jnp.float32),
        mesh=mesh,
        scratch_types=[
            pltpu.VMEM((128,), jnp.int32),
            pltpu.VMEM((128, D_EDGE), jnp.float32),
            pltpu.SemaphoreType.DMA,
        ],
        compiler_params=pltpu.CompilerParams(needs_layout_passes=False,
                                             use_tc_tiling_on_sc=False),
    )
    return f(eidx_flat, edge_features)


_COS_COEFFS = (1.0, -0.5, 0.0416666641831398, -0.0013888858957216144,
               2.4800418032100424e-05, -2.753243677489081e-07,
               2.058421877393357e-09, -9.662048938707812e-12)
_INV2PI = 0.15915494309189535
_TWOPI_HI = 6.28125
_TWOPI_LO = 0.0019353071795864769


def _fast_cos(x):
    n = jnp.floor(x * _INV2PI + 0.5)
    r = x - n * _TWOPI_HI
    r = r - n * _TWOPI_LO
    r2 = r * r
    acc = jnp.full_like(x, _COS_COEFFS[-1])
    for c in _COS_COEFFS[-2::-1]:
        acc = acc * r2 + c
    return acc


def _tc_attn_body(src_ref, nb_ref, ef_ref, dt_ref, nbr_ref, tw_ref, tb_ref,
                  wqm_ref, qb_ref, wkm_ref, wkp_ref, wke_ref,
                  wvm_ref, wvp_ref, wve_ref, wo_ref,
                  f1a_ref, f1c_ref, f1b_ref, f2w_ref, f2b_ref, out_ref):
    f32 = jnp.float32
    dot = functools.partial(jnp.dot, preferred_element_type=f32)
    tw = tw_ref[...]
    tb = tb_ref[...]
    src = src_ref[...]

    q = dot(src, wqm_ref[...]) + qb_ref[...]

    bf = jnp.bfloat16
    wkm = wkm_ref[...].astype(bf)
    wkp = wkp_ref[...].astype(bf)
    wke = wke_ref[...].astype(bf)
    wvm = wvm_ref[...].astype(bf)
    wvp = wvp_ref[...].astype(bf)
    wve = wve_ref[...].astype(bf)
    ks = []
    vs = []
    for n in range(NB):
        nb_n = nb_ref[n].astype(bf)
        ef_n = ef_ref[n].astype(bf)
        phi_n = _fast_cos(dt_ref[:, n:n + 1] * tw + tb).astype(bf)
        k_n = dot(nb_n, wkm) + dot(phi_n, wkp) + dot(ef_n, wke)
        v_n = dot(nb_n, wvm) + dot(phi_n, wvp) + dot(ef_n, wve)
        ks.append(k_n)
        vs.append(v_n)

    scale = 1.0 / (DH ** 0.5)
    outs = []
    for h in range(H):
        qh = q[:, h * DH:(h + 1) * DH]
        cols = []
        for n in range(NB):
            s_n = jnp.sum(qh * ks[n][:, h * DH:(h + 1) * DH], axis=1,
                          keepdims=True) * scale
            s_n = jnp.where(nbr_ref[:, n:n + 1] == 0, -1e9, s_n)
            cols.append(s_n)
        s = jnp.concatenate(cols, axis=1)
        s = s - jnp.max(s, axis=1, keepdims=True)
        e = jnp.exp(s)
        a = e / jnp.sum(e, axis=1, keepdims=True)
        o_h = jnp.zeros_like(qh)
        for n in range(NB):
            o_h = o_h + a[:, n:n + 1] * vs[n][:, h * DH:(h + 1) * DH]
        outs.append(o_h)
    att = jnp.concatenate(outs, axis=1)
    out = dot(att, wo_ref[...])
    merged = jnp.maximum(
        dot(out, f1a_ref[...]) + dot(src, f1c_ref[...]) + f1b_ref[...], 0.0)
    out_ref[...] = dot(merged, f2w_ref[...]) + f2b_ref[...]


def _tc_attn(src_feat, nb3, ef3, dt3, nbr3, time_w, time_b, Wq, qbias, Wk, Wv,
             Wo, fc1_w, fc1_b, fc2_w, fc2_b):
    BQ = 128
    grid = (NQ // BQ,)
    const = lambda shape: pl.BlockSpec(shape, lambda i: tuple(0 for _ in shape))
    in_specs = [
        pl.BlockSpec((BQ, D), lambda i: (i, 0)),
        pl.BlockSpec((NB, BQ, D), lambda i: (0, i, 0)),
        pl.BlockSpec((NB, BQ, D_EDGE), lambda i: (0, i, 0)),
        pl.BlockSpec((BQ, NB), lambda i: (i, 0)),
        pl.BlockSpec((BQ, NB), lambda i: (i, 0)),
        const((1, D)),
        const((1, D)),
        const((D, QDIM)), const((1, QDIM)),
        const((D, QDIM)), const((D, QDIM)), const((D_EDGE, QDIM)),
        const((D, QDIM)), const((D, QDIM)), const((D_EDGE, QDIM)),
        const((QDIM, QDIM)),
        const((QDIM, D)), const((D, D)), const((1, D)),
        const((D, D)), const((1, D)),
    ]
    return pl.pallas_call(
        _tc_attn_body,
        grid=grid,
        in_specs=in_specs,
        out_specs=pl.BlockSpec((BQ, D), lambda i: (i, 0)),
        out_shape=jax.ShapeDtypeStruct((NQ, D), jnp.float32),
    )(src_feat, nb3, ef3, dt3, nbr3,
      time_w.reshape(1, D), time_b.reshape(1, D),
      Wq[:D], qbias, Wk[:D], Wk[D:2 * D], Wk[2 * D:],
      Wv[:D], Wv[D:2 * D], Wv[2 * D:], Wo,
      fc1_w[:QDIM], fc1_w[QDIM:], fc1_b.reshape(1, D),
      fc2_w, fc2_b.reshape(1, D))


def kernel(node_features, edge_features, memory, time_w, time_b, msg_w1,
           msg_b1, msg_w2, msg_b2, gru_wih, gru_whh, gru_bih, gru_bhh, Wq, Wk,
           Wv, Wo, fc1_w, fc1_b, fc2_w, fc2_b, edge_times, neighbor_times,
           pending_msg_raw, source_nodes, destination_nodes, p_pos_nodes,
           p_neg_nodes, edge_idxs, neighbors, neighbor_edge_idxs,
           pending_msg_nodes):
    pending = pending_msg_nodes.astype(jnp.int32)

    h, nfp = _sc_gather_pending(memory, node_features, pending)
    upd, qbias = _tc_part1(pending_msg_raw, h, nfp, msg_w1, msg_b1, msg_w2,
                           msg_b2, gru_wih.T, gru_whh.T, gru_bih, gru_bhh,
                           time_b, Wq[D:])
    ext = _tc_ext(memory, node_features, upd)

    nodes = jnp.concatenate(
        [source_nodes, destination_nodes, p_pos_nodes, p_neg_nodes]
    ).astype(jnp.int32)
    nbr_flat = neighbors.T.reshape(-1).astype(jnp.int32)
    eidx_flat = neighbor_edge_idxs.T.reshape(-1).astype(jnp.int32)
    iota = jnp.arange(N_NODES, dtype=jnp.int32)

    src_feat, nb_flat = _sc_gather_main(ext, iota, pending, nodes, nbr_flat)
    ef_flat = _sc_gather_ef(eidx_flat, edge_features)

    ts = jnp.tile(edge_times, 4)
    dt2 = ts[:, None] - neighbor_times
    nb3 = nb_flat.reshape(NB, NQ, D)
    ef3 = ef_flat.reshape(NB, NQ, D_EDGE)
    nbr2 = neighbors.astype(jnp.int32)

    return _tc_attn(src_feat, nb3, ef3, dt2, nbr2, time_w, time_b,
                    Wq, qbias, Wk, Wv, Wo, fc1_w, fc1_b, fc2_w, fc2_b)


# speedup vs baseline: 1.4908x; 1.0690x over previous
import functools

import jax
import jax.numpy as jnp
from jax import lax
from jax.experimental import pallas as pl
from jax.experimental.pallas import tpu as pltpu
from jax.experimental.pallas import tpu_sc as plsc

N_NODES = 10000
D = 256
D_EDGE = 16
MEM = 256
B = 1024
NQ = 4096
NB = 20
H = 2
QDIM = 2 * D
DH = QDIM // H

NC = 2
NS = 16
NW = NC * NS
L = 16

EXT_ROWS = N_NODES + B


def _sc_pending_body(mem_hbm, nf_hbm, pend_hbm, h_out, nf_out, pend_v, buf, sem):
    wid = lax.axis_index("s") * NC + lax.axis_index("c")
    per_w = B // NW
    base = pl.multiple_of(wid * per_w, 8)
    pltpu.sync_copy(pend_hbm, pend_v)
    idx = pend_v.at[pl.ds(base, per_w)]
    pltpu.async_copy(mem_hbm.at[idx], buf, sem).wait()
    pltpu.sync_copy(buf, h_out.at[pl.ds(base, per_w)])
    pltpu.async_copy(nf_hbm.at[idx], buf, sem).wait()
    pltpu.sync_copy(buf, nf_out.at[pl.ds(base, per_w)])


def _sc_gather_pending(memory, node_features, pending):
    mesh = plsc.VectorSubcoreMesh(core_axis_name="c", subcore_axis_name="s")
    per_w = B // NW
    f = pl.kernel(
        _sc_pending_body,
        out_type=[
            jax.ShapeDtypeStruct((B, MEM), jnp.float32),
            jax.ShapeDtypeStruct((B, D), jnp.float32),
        ],
        mesh=mesh,
        scratch_types=[
            pltpu.VMEM((B,), jnp.int32),
            pltpu.VMEM((per_w, D), jnp.float32),
            pltpu.SemaphoreType.DMA,
        ],
        compiler_params=pltpu.CompilerParams(needs_layout_passes=False,
                                             use_tc_tiling_on_sc=True),
    )
    return f(memory, node_features, pending)


def _tc_part1_body(raw_ref, h_ref, nfp_ref, w1_ref, b1_ref, w2_ref, b2_ref,
                   wih_ref, whh_ref, bih_ref, bhh_ref, tb_ref, wqp_ref,
                   out_ref, qbias_ref):
    f32 = jnp.float32
    qbias_ref[...] = jnp.dot(jnp.cos(tb_ref[...]), wqp_ref[...],
                             preferred_element_type=f32)
    raw = raw_ref[...]
    hid = jnp.maximum(
        jnp.dot(raw, w1_ref[...], preferred_element_type=f32) + b1_ref[...], 0.0)
    msg = jnp.dot(hid, w2_ref[...], preferred_element_type=f32) + b2_ref[...]
    h = h_ref[...]
    gi = jnp.dot(msg, wih_ref[...], preferred_element_type=f32) + bih_ref[...]
    gh = jnp.dot(h, whh_ref[...], preferred_element_type=f32) + bhh_ref[...]
    i_r, i_z, i_n = gi[:, :MEM], gi[:, MEM:2 * MEM], gi[:, 2 * MEM:]
    h_r, h_z, h_n = gh[:, :MEM], gh[:, MEM:2 * MEM], gh[:, 2 * MEM:]
    r = jax.nn.sigmoid(i_r + h_r)
    z = jax.nn.sigmoid(i_z + h_z)
    n = jnp.tanh(i_n + r * h_n)
    h_new = (1.0 - z) * n + z * h
    out_ref[...] = h_new + nfp_ref[...]


def _tc_part1(raw, h, nfp, w1, b1, w2, b2, wih_t, whh_t, bih, bhh, tb, wqp):
    return pl.pallas_call(
        _tc_part1_body,
        out_shape=[jax.ShapeDtypeStruct((B, D), jnp.float32),
                   jax.ShapeDtypeStruct((1, QDIM), jnp.float32)],
    )(raw, h, nfp, w1, b1.reshape(1, -1), w2, b2.reshape(1, -1),
      wih_t, whh_t, bih.reshape(1, -1), bhh.reshape(1, -1),
      tb.reshape(1, -1), wqp)


_ADD_BLK = 1000
_N_ADD = N_NODES // _ADD_BLK


def _tc_ext_body(a_ref, b_ref, u_ref, o_ref):
    i = pl.program_id(0)

    @pl.when(i < _N_ADD)
    def _():
        o_ref[...] = a_ref[...] + b_ref[...]

    @pl.when(i >= _N_ADD)
    def _():
        o_ref[...] = u_ref[...]


def _tc_ext(memory, node_features, upd):
    nsteps = _N_ADD + (B + _ADD_BLK - 1) // _ADD_BLK
    return pl.pallas_call(
        _tc_ext_body,
        grid=(nsteps,),
        in_specs=[
            pl.BlockSpec((_ADD_BLK, D), lambda i: (jnp.minimum(i, _N_ADD - 1), 0)),
            pl.BlockSpec((_ADD_BLK, D), lambda i: (jnp.minimum(i, _N_ADD - 1), 0)),
            pl.BlockSpec((_ADD_BLK, D), lambda i: (jnp.maximum(i - _N_ADD, 0), 0)),
        ],
        out_specs=pl.BlockSpec((_ADD_BLK, D), lambda i: (i, 0)),
        out_shape=jax.ShapeDtypeStruct((EXT_ROWS, D), jnp.float32),
    )(memory, node_features, upd)


def _sc_main_body(nqh, ext_hbm, iota_hbm, pend_hbm, nodes_hbm, nbr_hbm,
                  src_out, nb_out,
                  t_v, pend_v, idx_v, tidx_v, rowbuf, sem):
    wid = lax.axis_index("s") * NC + lax.axis_index("c")
    lanes = lax.iota(jnp.int32, L)

    pltpu.sync_copy(iota_hbm, t_v)
    pltpu.sync_copy(pend_hbm, pend_v.at[pl.ds(0, B)])
    pend_v[pl.ds(B, L)] = jnp.full((L,), -1, jnp.int32)
    for i in range(B // L):
        idx = pend_v[pl.ds(i * L, L)]
        nxt = plsc.load_gather(pend_v, [lanes + (i * L + 1)])
        keep = idx != nxt
        vals = lanes + (N_NODES + i * L)
        plsc.store_scatter(t_v, [idx], vals, mask=keep)

    spw = nqh // NW
    sbase = pl.multiple_of(wid * spw, 8)
    pltpu.sync_copy(nodes_hbm.at[pl.ds(sbase, spw)], idx_v.at[pl.ds(0, spw)])
    for j in range(spw // L):
        v = idx_v[pl.ds(j * L, L)]
        tidx_v[pl.ds(j * L, L)] = plsc.load_gather(t_v, [v])
    pltpu.async_copy(ext_hbm.at[tidx_v.at[pl.ds(0, spw)]],
                     rowbuf.at[pl.ds(0, spw)], sem).wait()
    pltpu.sync_copy(rowbuf.at[pl.ds(0, spw)], src_out.at[pl.ds(sbase, spw)])

    per_w = (nqh * NB) // NW
    nchunks = per_w // 128

    def nb_chunk(c, _):
        base = pl.multiple_of(wid * per_w + c * 128, 8)
        pltpu.sync_copy(nbr_hbm.at[pl.ds(base, 128)], idx_v)
        for j in range(128 // L):
            v = idx_v[pl.ds(j * L, L)]
            tidx_v[pl.ds(j * L, L)] = plsc.load_gather(t_v, [v])
        pltpu.async_copy(ext_hbm.at[tidx_v], rowbuf, sem).wait()
        pltpu.sync_copy(rowbuf, nb_out.at[pl.ds(base, 128)])
        return 0

    lax.fori_loop(0, nchunks, nb_chunk, 0)


def _sc_gather_main(ext, iota, pending, nodes, nbr_flat):
    nqh = nodes.shape[0]
    mesh = plsc.VectorSubcoreMesh(core_axis_name="c", subcore_axis_name="s")
    f = pl.kernel(
        functools.partial(_sc_main_body, nqh),
        out_type=[
            jax.ShapeDtypeStruct((nqh, D), jnp.float32),
            jax.ShapeDtypeStruct((nqh * NB, D), jnp.float32),
        ],
        mesh=mesh,
        scratch_types=[
            pltpu.VMEM((N_NODES,), jnp.int32),
            pltpu.VMEM((B + L,), jnp.int32),
            pltpu.VMEM((128,), jnp.int32),
            pltpu.VMEM((128,), jnp.int32),
            pltpu.VMEM((128, D), jnp.float32),
            pltpu.SemaphoreType.DMA,
        ],
        compiler_params=pltpu.CompilerParams(needs_layout_passes=False,
                                             use_tc_tiling_on_sc=True),
    )
    return f(ext, iota, pending, nodes, nbr_flat)


def _sc_ef_body(nqh, eidx_hbm, ef_hbm, ef_out, idx_v, efbuf, sem):
    wid = lax.axis_index("s") * NC + lax.axis_index("c")
    per_w = (nqh * NB) // NW

    def ef_chunk(c, _):
        base = pl.multiple_of(wid * per_w + c * 128, 8)
        pltpu.sync_copy(eidx_hbm.at[pl.ds(base, 128)], idx_v)
        pltpu.async_copy(ef_hbm.at[idx_v], efbuf, sem).wait()
        pltpu.sync_copy(efbuf, ef_out.at[pl.ds(base, 128)])
        return 0

    lax.fori_loop(0, per_w // 128, ef_chunk, 0)


def _sc_gather_ef(eidx_flat, edge_features):
    nqh = eidx_flat.shape[0] // NB
    mesh = plsc.VectorSubcoreMesh(core_axis_name="c", subcore_axis_name="s")
    f = pl.kernel(
        functools.partial(_sc_ef_body, nqh),
        out_type=jax.ShapeDtypeStruct((nqh * NB, D_EDGE), jnp.float32),
        mesh=mesh,
        scratch_types=[
            pltpu.VMEM((128,), jnp.int32),
            pltpu.VMEM((128, D_EDGE), jnp.float32),
            pltpu.SemaphoreType.DMA,
        ],
        compiler_params=pltpu.CompilerParams(needs_layout_passes=False,
                                             use_tc_tiling_on_sc=False),
    )
    return f(eidx_flat, edge_features)


_COS_COEFFS = (1.0, -0.5, 0.0416666641831398, -0.0013888858957216144,
               2.4800418032100424e-05, -2.753243677489081e-07,
               2.058421877393357e-09, -9.662048938707812e-12)
_INV2PI = 0.15915494309189535
_TWOPI_HI = 6.28125
_TWOPI_LO = 0.0019353071795864769


def _fast_cos(x):
    n = jnp.floor(x * _INV2PI + 0.5)
    r = x - n * _TWOPI_HI
    r = r - n * _TWOPI_LO
    r2 = r * r
    acc = jnp.full_like(x, _COS_COEFFS[-1])
    for c in _COS_COEFFS[-2::-1]:
        acc = acc * r2 + c
    return acc


def _tc_attn_body(src_ref, nb_ref, ef_ref, dt_ref, nbr_ref, tw_ref, tb_ref,
                  wqm_ref, qb_ref, wkm_ref, wkp_ref, wke_ref,
                  wvm_ref, wvp_ref, wve_ref, wo_ref,
                  f1a_ref, f1c_ref, f1b_ref, f2w_ref, f2b_ref, out_ref):
    f32 = jnp.float32
    dot = functools.partial(jnp.dot, preferred_element_type=f32)
    tw = tw_ref[...]
    tb = tb_ref[...]
    src = src_ref[...]

    q = dot(src, wqm_ref[...]) + qb_ref[...]

    bf = jnp.bfloat16
    wkm = wkm_ref[...].astype(bf)
    wkp = wkp_ref[...].astype(bf)
    wke = wke_ref[...].astype(bf)
    wvm = wvm_ref[...].astype(bf)
    wvp = wvp_ref[...].astype(bf)
    wve = wve_ref[...].astype(bf)
    ks = []
    vs = []
    for n in range(NB):
        nb_n = nb_ref[n].astype(bf)
        ef_n = ef_ref[n].astype(bf)
        phi_n = _fast_cos(dt_ref[:, n:n + 1] * tw + tb).astype(bf)
        k_n = dot(nb_n, wkm) + dot(phi_n, wkp) + dot(ef_n, wke)
        v_n = dot(nb_n, wvm) + dot(phi_n, wvp) + dot(ef_n, wve)
        ks.append(k_n)
        vs.append(v_n)

    scale = 1.0 / (DH ** 0.5)
    outs = []
    for h in range(H):
        qh = q[:, h * DH:(h + 1) * DH]
        cols = []
        for n in range(NB):
            s_n = jnp.sum(qh * ks[n][:, h * DH:(h + 1) * DH], axis=1,
                          keepdims=True) * scale
            s_n = jnp.where(nbr_ref[:, n:n + 1] == 0, -1e9, s_n)
            cols.append(s_n)
        s = jnp.concatenate(cols, axis=1)
        s = s - jnp.max(s, axis=1, keepdims=True)
        e = jnp.exp(s)
        a = e / jnp.sum(e, axis=1, keepdims=True)
        o_h = jnp.zeros_like(qh)
        for n in range(NB):
            o_h = o_h + a[:, n:n + 1] * vs[n][:, h * DH:(h + 1) * DH]
        outs.append(o_h)
    att = jnp.concatenate(outs, axis=1)
    out = dot(att, wo_ref[...])
    merged = jnp.maximum(
        dot(out, f1a_ref[...]) + dot(src, f1c_ref[...]) + f1b_ref[...], 0.0)
    out_ref[...] = dot(merged, f2w_ref[...]) + f2b_ref[...]


def _tc_attn(src_feat, nb3, ef3, dt3, nbr3, time_w, time_b, Wq, qbias, Wk, Wv,
             Wo, fc1_w, fc1_b, fc2_w, fc2_b):
    nqh = src_feat.shape[0]
    BQ = 128
    grid = (nqh // BQ,)
    const = lambda shape: pl.BlockSpec(shape, lambda i: tuple(0 for _ in shape))
    in_specs = [
        pl.BlockSpec((BQ, D), lambda i: (i, 0)),
        pl.BlockSpec((NB, BQ, D), lambda i: (0, i, 0)),
        pl.BlockSpec((NB, BQ, D_EDGE), lambda i: (0, i, 0)),
        pl.BlockSpec((BQ, NB), lambda i: (i, 0)),
        pl.BlockSpec((BQ, NB), lambda i: (i, 0)),
        const((1, D)),
        const((1, D)),
        const((D, QDIM)), const((1, QDIM)),
        const((D, QDIM)), const((D, QDIM)), const((D_EDGE, QDIM)),
        const((D, QDIM)), const((D, QDIM)), const((D_EDGE, QDIM)),
        const((QDIM, QDIM)),
        const((QDIM, D)), const((D, D)), const((1, D)),
        const((D, D)), const((1, D)),
    ]
    return pl.pallas_call(
        _tc_attn_body,
        grid=grid,
        in_specs=in_specs,
        out_specs=pl.BlockSpec((BQ, D), lambda i: (i, 0)),
        out_shape=jax.ShapeDtypeStruct((nqh, D), jnp.float32),
    )(src_feat, nb3, ef3, dt3, nbr3,
      time_w.reshape(1, D), time_b.reshape(1, D),
      Wq[:D], qbias, Wk[:D], Wk[D:2 * D], Wk[2 * D:],
      Wv[:D], Wv[D:2 * D], Wv[2 * D:], Wo,
      fc1_w[:QDIM], fc1_w[QDIM:], fc1_b.reshape(1, D),
      fc2_w, fc2_b.reshape(1, D))


def kernel(node_features, edge_features, memory, time_w, time_b, msg_w1,
           msg_b1, msg_w2, msg_b2, gru_wih, gru_whh, gru_bih, gru_bhh, Wq, Wk,
           Wv, Wo, fc1_w, fc1_b, fc2_w, fc2_b, edge_times, neighbor_times,
           pending_msg_raw, source_nodes, destination_nodes, p_pos_nodes,
           p_neg_nodes, edge_idxs, neighbors, neighbor_edge_idxs,
           pending_msg_nodes):
    pending = pending_msg_nodes.astype(jnp.int32)

    h, nfp = _sc_gather_pending(memory, node_features, pending)
    upd, qbias = _tc_part1(pending_msg_raw, h, nfp, msg_w1, msg_b1, msg_w2,
                           msg_b2, gru_wih.T, gru_whh.T, gru_bih, gru_bhh,
                           time_b, Wq[D:])
    ext = _tc_ext(memory, node_features, upd)

    nodes = jnp.concatenate(
        [source_nodes, destination_nodes, p_pos_nodes, p_neg_nodes]
    ).astype(jnp.int32)
    nbrT = neighbors.T.astype(jnp.int32)
    eidxT = neighbor_edge_idxs.T.astype(jnp.int32)
    iota = jnp.arange(N_NODES, dtype=jnp.int32)
    ts = jnp.tile(edge_times, 4)
    dt2 = ts[:, None] - neighbor_times
    nbr2 = neighbors.astype(jnp.int32)

    NH = NQ // 2
    embs = []
    for s in (0, NH):
        nodes_h = nodes[s:s + NH]
        nbr_flat_h = nbrT[:, s:s + NH].reshape(-1)
        eidx_flat_h = eidxT[:, s:s + NH].reshape(-1)
        src_feat, nb_flat = _sc_gather_main(ext, iota, pending, nodes_h,
                                            nbr_flat_h)
        ef_flat = _sc_gather_ef(eidx_flat_h, edge_features)
        nb3 = nb_flat.reshape(NB, NH, D)
        ef3 = ef_flat.reshape(NB, NH, D_EDGE)
        embs.append(_tc_attn(src_feat, nb3, ef3, dt2[s:s + NH],
                             nbr2[s:s + NH], time_w, time_b, Wq, qbias,
                             Wk, Wv, Wo, fc1_w, fc1_b, fc2_w, fc2_b))
    return jnp.concatenate(embs, axis=0)
